# Initial kernel scaffold; baseline (speedup 1.0000x reference)
import jax
import jax.numpy as jnp
from jax.experimental import pallas as pl


def kernel(x, y, W, b, encoding):
    raise NotImplementedError("write your pallas kernel here")


# speedup vs baseline: 5.0723x; 5.0723x over previous
import functools

import jax
import jax.numpy as jnp
from jax import lax
from jax.experimental import pallas as pl
from jax.experimental.pallas import tpu as pltpu
from jax.experimental.pallas import tpu_sc as plsc

_D = 128
_MX = 256
_MY = 256
_B = 4096
_L = 50
_N = _B * _L
_WIN = 128
_NROW = _N // _D


def _tc_prep(xr, yr, exh, eyh, e00, W, b):
    def body(xr_ref, yr_ref, exh_ref, eyh_ref, e00_ref, w_ref, b_ref,
             axb_ref, ay_ref, ix_ref, iy_ref):
        wm = w_ref[...]
        dn = (((1,), (1,)), ((), ()))
        axb_ref[...] = lax.dot_general(
            exh_ref[...], wm, dn, preferred_element_type=jnp.float32
        ) + b_ref[...]
        ay_ref[...] = lax.dot_general(
            eyh_ref[...] - e00_ref[...], wm, dn,
            preferred_element_type=jnp.float32)
        ix_ref[...] = (xr_ref[...] * (_MX - 1.0)).astype(jnp.int32)
        iy_ref[...] = (yr_ref[...] * (_MY - 1.0)).astype(jnp.int32)

    return pl.pallas_call(
        body,
        out_shape=(
            jax.ShapeDtypeStruct((_MX, _D), jnp.float32),
            jax.ShapeDtypeStruct((_MY, _D), jnp.float32),
            jax.ShapeDtypeStruct((_NROW, _D), jnp.int32),
            jax.ShapeDtypeStruct((_NROW, _D), jnp.int32),
        ),
    )(xr, yr, exh, eyh, e00, W, b)


def _sc_lookup(axb, ay, ixf, iyf):
    mesh = plsc.VectorSubcoreMesh(core_axis_name="core",
                                  subcore_axis_name="subcore")

    @functools.partial(
        pl.kernel,
        out_type=jax.ShapeDtypeStruct((_N, _D), jnp.float32),
        mesh=mesh,
        scratch_types=[pltpu.VMEM((_WIN, _D), jnp.float32)],
    )
    def kern(axb_hbm, ay_hbm, ix_hbm, iy_hbm, o_hbm, rb_vmem):
        def body(ix_vmem, iy_vmem, o_vmem):
            pltpu.sync_copy(axb_hbm.at[ix_vmem.at[0]], o_vmem)
            pltpu.sync_copy(ay_hbm.at[iy_vmem.at[0]], rb_vmem)

            @pl.loop(0, _WIN)
            def _row(r):
                for c in range(0, _D, 16):
                    slc = (pl.ds(r, 1), pl.ds(c, 16))
                    o_vmem.at[slc][...] = jnp.maximum(
                        o_vmem.at[slc][...] + rb_vmem.at[slc][...], 0.0)

        pltpu.emit_pipeline(
            body,
            grid=(_N // _WIN,),
            in_specs=[pl.BlockSpec((1, _WIN), lambda i: (0, i)),
                      pl.BlockSpec((1, _WIN), lambda i: (0, i))],
            out_specs=[pl.BlockSpec((_WIN, _D), lambda i: (i, 0))],
            core_axis_name=("core", "subcore"),
            dimension_semantics=(pltpu.PARALLEL,),
        )(ix_hbm, iy_hbm, o_hbm)

    return kern(axb, ay, ixf, iyf)


def kernel(x, y, W, b, encoding):
    exh = encoding[:, 0, :]
    eyh = encoding[0, :, :]
    e00 = encoding[0:1, 0, :]
    xr = x.reshape(_NROW, _D)
    yr = y.reshape(_NROW, _D)
    axb, ay, ix, iy = _tc_prep(xr, yr, exh, eyh, e00, W, b.reshape(1, _D))
    out = _sc_lookup(axb, ay, ix.reshape(1, _N), iy.reshape(1, _N))
    return out.reshape(_B, _L, _D)
